# baseline (speedup 1.0000x reference)
import functools

import jax
import jax.numpy as jnp
from jax.experimental import pallas as pl


def _leaky(x):
    return jnp.where(x >= 0, x, 0.2 * x)


def _conv_body(x0_ref, x1_ref, x2_ref, w_ref, b_ref, o_ref,
               *, W, Wp, th, act, scale):
    cb = w_ref.shape[-1]
    cin = w_ref.shape[-2]
    xs = [x0_ref, x1_ref, x2_ref]
    ts = []
    for dx in range(3):
        acc = jnp.zeros((th * Wp, cb), jnp.float32)
        for dy in range(3):
            acc = acc + jnp.dot(xs[dy][0].reshape(th * Wp, cin), w_ref[dy, dx],
                                preferred_element_type=jnp.float32)
        ts.append(acc.reshape(th, Wp, cb))
    out = ts[0][:, 0:W] + ts[1][:, 1:W + 1] + ts[2][:, 2:W + 2]
    out = out + b_ref[0][None, None, :]
    if act == 'leaky':
        out = _leaky(out)
    if scale != 1.0:
        out = out * scale
    o_ref[0] = out


def _round_up(v, m):
    return ((v + m - 1) // m) * m


def conv3x3_p(x, w, b, mode, act=None, scale=1.0, cb=None, th=16):
    B, H, W, Cin = x.shape
    Cout = w.shape[0]
    th = min(th, H)
    if mode == 'zero':
        xp = jnp.pad(x, ((0, 0), (1, 1), (1, 1), (0, 0)))
    else:
        xp = jnp.pad(x, ((0, 0), (1, 1), (1, 1), (0, 0)), mode=mode)
    Wp = _round_up(W + 2, 8)
    xp = jnp.pad(xp, ((0, 0), (0, 0), (0, Wp - (W + 2)), (0, 0)))

    co_pad = _round_up(Cout, 128)
    if cb is None:
        cb = 128
    n_co = co_pad // cb
    n_h = H // th
    wt = jnp.transpose(w, (2, 3, 1, 0))
    wt = jnp.pad(wt, ((0, 0), (0, 0), (0, 0), (0, co_pad - Cout)))
    bp = jnp.pad(b, (0, co_pad - Cout)).reshape(1, co_pad)
    v0, v1, v2 = xp[:, 0:H], xp[:, 1:H + 1], xp[:, 2:H + 2]

    body = functools.partial(_conv_body, W=W, Wp=Wp, th=th,
                             act=act, scale=scale)
    xspec = pl.BlockSpec((1, th, Wp, Cin), lambda ci, hi, bi: (bi, hi, 0, 0))
    out = pl.pallas_call(
        body,
        grid=(n_co, n_h, B),
        in_specs=[
            xspec, xspec, xspec,
            pl.BlockSpec((3, 3, Cin, cb), lambda ci, hi, bi: (0, 0, 0, ci)),
            pl.BlockSpec((1, cb), lambda ci, hi, bi: (0, ci)),
        ],
        out_specs=pl.BlockSpec((1, th, W, cb),
                               lambda ci, hi, bi: (bi, hi, 0, ci)),
        out_shape=jax.ShapeDtypeStruct((B, H, W, co_pad), jnp.float32),
    )(v0, v1, v2, wt, bp)
    return out[..., :Cout]


def _up2(x):
    return jnp.repeat(jnp.repeat(x, 2, axis=1), 2, axis=2)


def _maxpool(x, k):
    return jax.lax.reduce_window(x, -jnp.inf, jax.lax.max,
                                 (1, 1, k, k), (1, 1, 1, 1), 'SAME')


def _iwt_haar(ll, h):
    lh, hl, hh = h[:, :, 0], h[:, :, 1], h[:, :, 2]
    x00 = (ll - lh - hl + hh) * 0.5
    x01 = (ll - lh + hl - hh) * 0.5
    x10 = (ll + lh - hl - hh) * 0.5
    x11 = (ll + lh + hl + hh) * 0.5
    B, C, H, W = ll.shape
    out = jnp.zeros((B, C, 2 * H, 2 * W), ll.dtype)
    out = out.at[:, :, 0::2, 0::2].set(x00)
    out = out.at[:, :, 0::2, 1::2].set(x01)
    out = out.at[:, :, 1::2, 0::2].set(x10)
    out = out.at[:, :, 1::2, 1::2].set(x11)
    return out


def _nhwc(x):
    return jnp.transpose(x, (0, 2, 3, 1))


def _nchw(x):
    return jnp.transpose(x, (0, 3, 1, 2))


def kernel(x_block_0, x_block_1, x_block_2, x_block_3,
           c2w, c2b, u1w, u1b, w1llw, w1llb, w1w, w1b,
           u2w, u2b, w2w, w2b, u3w, u3b, w3w, w3b):
    thresh_ratio = 0.1
    xb4, xb3, xb2, xb1 = x_block_0, x_block_1, x_block_2, x_block_3
    x1 = _nhwc(xb1)
    x_d0 = conv3x3_p(x1, c2w, c2b, 'edge')
    cat1 = jnp.concatenate([_up2(x_d0), _nhwc(xb2)], -1)
    x_d1 = conv3x3_p(cat1, u1w, u1b, 'reflect', act='leaky')
    ll_n = conv3x3_p(x_d1, w1llw, w1llb, 'edge', scale=8.0)
    disp3 = _nchw(ll_n) / 8.0
    h_n = conv3x3_p(x_d1, w1w, w1b, 'zero', scale=4.0)
    ll = _nchw(ll_n)
    h = _nchw(h_n)[:, None]
    ll = _iwt_haar(ll, h)
    disp2 = ll / 4.0

    thresh = (ll.max() - ll.min()) * thresh_ratio
    mask = (jnp.abs(h).max(axis=2) > thresh).astype(jnp.float32)
    up_mask = (_maxpool(mask, 5) > 0).astype(jnp.float32)
    conva_mask = (_maxpool(_up2_nchw(mask), 5) > 0).astype(jnp.float32)
    wave_mask = (_maxpool(_up2_nchw(mask), 3) > 0).astype(jnp.float32)
    wavelet_mask = _up2_nchw(mask)

    xv = x_d1 * _nhwc(up_mask)
    cat2 = jnp.concatenate([_up2(xv), _nhwc(xb3)], -1) * _nhwc(conva_mask)
    xv = conv3x3_p(cat2, u2w, u2b, 'reflect', act='leaky') * _nhwc(wave_mask)
    h2 = conv3x3_p(xv, w2w, w2b, 'zero', scale=2.0) * _nhwc(wavelet_mask)
    h = _nchw(h2)[:, None]
    ll = _iwt_haar(ll, wavelet_mask[:, :, None] * h)
    disp1 = ll / 2.0

    thresh = (ll.max() - ll.min()) * thresh_ratio
    mask = (jnp.abs(h).max(axis=2) > thresh).astype(jnp.float32)
    up_mask = (_maxpool(mask, 5) > 0).astype(jnp.float32)
    conva_mask = (_maxpool(_up2_nchw(mask), 5) > 0).astype(jnp.float32)
    wave_mask = (_maxpool(_up2_nchw(mask), 3) > 0).astype(jnp.float32)
    wavelet_mask = _up2_nchw(mask)

    xv = xv * _nhwc(up_mask)
    cat3 = jnp.concatenate([_up2(xv), _nhwc(xb4)], -1) * _nhwc(conva_mask)
    xv = conv3x3_p(cat3, u3w, u3b, 'reflect', act='leaky') * _nhwc(wave_mask)
    h3 = conv3x3_p(xv, w3w, w3b, 'zero') * _nhwc(wavelet_mask)
    h = _nchw(h3)[:, None]
    ll = _iwt_haar(ll, wavelet_mask[:, :, None] * h)
    disp0 = ll
    return disp3, disp2, disp1, disp0


def _up2_nchw(x):
    return jnp.repeat(jnp.repeat(x, 2, axis=2), 2, axis=3)


# speedup vs baseline: 1.0737x; 1.0737x over previous
import functools

import jax
import jax.numpy as jnp
from jax.experimental import pallas as pl


def _leaky(x):
    return jnp.where(x >= 0, x, 0.2 * x)


def _conv_body_impl(x_ref, w_ref, b_ref, o_ref, *, H, W, Wp, th, act, scale,
                    n_ci, ci_axis):
    cb = w_ref.shape[-1]
    cib = w_ref.shape[-2]
    nb = x_ref.shape[0]
    ci = pl.program_id(ci_axis)
    for b in range(nb):
        for h0 in range(0, H, th):
            ts = []
            for dx in range(3):
                acc = jnp.zeros((th * Wp, cb), jnp.float32)
                for dy in range(3):
                    xs = x_ref[b, h0 + dy:h0 + dy + th]
                    acc = acc + jnp.dot(xs.reshape(th * Wp, cib),
                                        w_ref[dy, dx],
                                        preferred_element_type=jnp.float32)
                ts.append(acc.reshape(th, Wp, cb))
            out = ts[0][:, 0:W] + ts[1][:, 1:W + 1] + ts[2][:, 2:W + 2]
            if n_ci == 1:
                out = out + b_ref[0][None, None, :]
                if act == 'leaky':
                    out = _leaky(out)
                if scale != 1.0:
                    out = out * scale
                o_ref[b, h0:h0 + th] = out
            else:
                @pl.when(ci == 0)
                def _init():
                    o_ref[b, h0:h0 + th] = out

                @pl.when(ci > 0)
                def _accum():
                    o_ref[b, h0:h0 + th] += out
    if n_ci > 1:
        @pl.when(ci == n_ci - 1)
        def _epilogue():
            for b in range(nb):
                y = o_ref[b] + b_ref[0][None, None, :]
                if act == 'leaky':
                    y = _leaky(y)
                if scale != 1.0:
                    y = y * scale
                o_ref[b] = y


def _round_up(v, m):
    return ((v + m - 1) // m) * m


def conv3x3_p(x, w, b, mode, act=None, scale=1.0, cb=128, cib=None, th=16,
              batch_in_grid=False):
    B, H, W, Cin = x.shape
    Cout = w.shape[0]
    th = min(th, H)
    if mode == 'zero':
        xp = jnp.pad(x, ((0, 0), (1, 1), (1, 1), (0, 0)))
    else:
        xp = jnp.pad(x, ((0, 0), (1, 1), (1, 1), (0, 0)), mode=mode)
    Wp = _round_up(W + 2, 8)
    co_pad = _round_up(Cout, cb)
    n_co = co_pad // cb
    if cib is None:
        cib = _round_up(Cin, 128)
    ci_pad = _round_up(Cin, cib)
    n_ci = ci_pad // cib
    xp = jnp.pad(xp, ((0, 0), (0, 0), (0, Wp - (W + 2)),
                      (0, ci_pad - Cin)))
    wt = jnp.transpose(w, (2, 3, 1, 0))
    wt = jnp.pad(wt, ((0, 0), (0, 0), (0, ci_pad - Cin), (0, co_pad - Cout)))
    bp = jnp.pad(b, (0, co_pad - Cout)).reshape(1, co_pad)

    if batch_in_grid:
        grid = (B, n_co, n_ci)
        nb = 1
        xmap = lambda bi, co, ci: (bi, 0, 0, ci)
        wmap = lambda bi, co, ci: (0, 0, ci, co)
        bmap = lambda bi, co, ci: (0, co)
        omap = lambda bi, co, ci: (bi, 0, 0, co)
        ci_axis = 2
    else:
        grid = (n_co, n_ci)
        nb = B
        xmap = lambda co, ci: (0, 0, 0, ci)
        wmap = lambda co, ci: (0, 0, ci, co)
        bmap = lambda co, ci: (0, co)
        omap = lambda co, ci: (0, 0, 0, co)
        ci_axis = 1

    body = functools.partial(_conv_body_impl, H=H, W=W, Wp=Wp, th=th,
                             act=act, scale=scale, n_ci=n_ci, ci_axis=ci_axis)
    out = pl.pallas_call(
        body,
        grid=grid,
        in_specs=[
            pl.BlockSpec((nb, H + 2, Wp, cib), xmap),
            pl.BlockSpec((3, 3, cib, cb), wmap),
            pl.BlockSpec((1, cb), bmap),
        ],
        out_specs=pl.BlockSpec((nb, H, W, cb), omap),
        out_shape=jax.ShapeDtypeStruct((B, H, W, co_pad), jnp.float32),
    )(xp, wt, bp)
    return out[..., :Cout]


def _up2(x):
    return jnp.repeat(jnp.repeat(x, 2, axis=1), 2, axis=2)


def _maxpool(x, k):
    return jax.lax.reduce_window(x, -jnp.inf, jax.lax.max,
                                 (1, 1, k, k), (1, 1, 1, 1), 'SAME')


def _iwt_haar(ll, h):
    lh, hl, hh = h[:, :, 0], h[:, :, 1], h[:, :, 2]
    x00 = (ll - lh - hl + hh) * 0.5
    x01 = (ll - lh + hl - hh) * 0.5
    x10 = (ll + lh - hl - hh) * 0.5
    x11 = (ll + lh + hl + hh) * 0.5
    B, C, H, W = ll.shape
    out = jnp.zeros((B, C, 2 * H, 2 * W), ll.dtype)
    out = out.at[:, :, 0::2, 0::2].set(x00)
    out = out.at[:, :, 0::2, 1::2].set(x01)
    out = out.at[:, :, 1::2, 0::2].set(x10)
    out = out.at[:, :, 1::2, 1::2].set(x11)
    return out


def _nhwc(x):
    return jnp.transpose(x, (0, 2, 3, 1))


def _nchw(x):
    return jnp.transpose(x, (0, 3, 1, 2))


def kernel(x_block_0, x_block_1, x_block_2, x_block_3,
           c2w, c2b, u1w, u1b, w1llw, w1llb, w1w, w1b,
           u2w, u2b, w2w, w2b, u3w, u3b, w3w, w3b):
    thresh_ratio = 0.1
    xb4, xb3, xb2, xb1 = x_block_0, x_block_1, x_block_2, x_block_3
    x1 = _nhwc(xb1)
    x_d0 = conv3x3_p(x1, c2w, c2b, 'edge', cb=384, cib=768)
    cat1 = jnp.concatenate([_up2(x_d0), _nhwc(xb2)], -1)
    x_d1 = conv3x3_p(cat1, u1w, u1b, 'reflect', act='leaky',
                     cb=256, cib=768)
    ll_n = conv3x3_p(x_d1, w1llw, w1llb, 'edge', scale=8.0)
    disp3 = _nchw(ll_n) / 8.0
    h_n = conv3x3_p(x_d1, w1w, w1b, 'zero', scale=4.0)
    ll = _nchw(ll_n)
    h = _nchw(h_n)[:, None]
    ll = _iwt_haar(ll, h)
    disp2 = ll / 4.0

    thresh = (ll.max() - ll.min()) * thresh_ratio
    mask = (jnp.abs(h).max(axis=2) > thresh).astype(jnp.float32)
    up_mask = (_maxpool(mask, 5) > 0).astype(jnp.float32)
    conva_mask = (_maxpool(_up2_nchw(mask), 5) > 0).astype(jnp.float32)
    wave_mask = (_maxpool(_up2_nchw(mask), 3) > 0).astype(jnp.float32)
    wavelet_mask = _up2_nchw(mask)

    xv = x_d1 * _nhwc(up_mask)
    cat2 = jnp.concatenate([_up2(xv), _nhwc(xb3)], -1) * _nhwc(conva_mask)
    xv = conv3x3_p(cat2, u2w, u2b, 'reflect', act='leaky',
                   cb=128, cib=384) * _nhwc(wave_mask)
    h2 = conv3x3_p(xv, w2w, w2b, 'zero', scale=2.0) * _nhwc(wavelet_mask)
    h = _nchw(h2)[:, None]
    ll = _iwt_haar(ll, wavelet_mask[:, :, None] * h)
    disp1 = ll / 2.0

    thresh = (ll.max() - ll.min()) * thresh_ratio
    mask = (jnp.abs(h).max(axis=2) > thresh).astype(jnp.float32)
    up_mask = (_maxpool(mask, 5) > 0).astype(jnp.float32)
    conva_mask = (_maxpool(_up2_nchw(mask), 5) > 0).astype(jnp.float32)
    wave_mask = (_maxpool(_up2_nchw(mask), 3) > 0).astype(jnp.float32)
    wavelet_mask = _up2_nchw(mask)

    xv = xv * _nhwc(up_mask)
    cat3 = jnp.concatenate([_up2(xv), _nhwc(xb4)], -1) * _nhwc(conva_mask)
    xv = conv3x3_p(cat3, u3w, u3b, 'reflect', act='leaky',
                   cb=128, cib=128, batch_in_grid=True) * _nhwc(wave_mask)
    h3 = conv3x3_p(xv, w3w, w3b, 'zero') * _nhwc(wavelet_mask)
    h = _nchw(h3)[:, None]
    ll = _iwt_haar(ll, wavelet_mask[:, :, None] * h)
    disp0 = ll
    return disp3, disp2, disp1, disp0


def _up2_nchw(x):
    return jnp.repeat(jnp.repeat(x, 2, axis=2), 2, axis=3)


# speedup vs baseline: 1.0755x; 1.0017x over previous
import functools

import jax
import jax.numpy as jnp
from jax.experimental import pallas as pl


def _leaky(x):
    return jnp.where(x >= 0, x, 0.2 * x)


def _conv_body_impl(x_ref, w_ref, b_ref, o_ref, *, H, W, Wp, th, act, scale,
                    n_ci, ci_axis):
    cb = w_ref.shape[-1]
    cib = w_ref.shape[-2]
    nb = x_ref.shape[0]
    ci = pl.program_id(ci_axis)
    for b in range(nb):
        for h0 in range(0, H, th):
            ts = []
            for dx in range(3):
                acc = jnp.zeros((th * Wp, cb), jnp.float32)
                for dy in range(3):
                    xs = x_ref[b, h0 + dy:h0 + dy + th]
                    acc = acc + jnp.dot(xs.reshape(th * Wp, cib),
                                        w_ref[dy, dx],
                                        preferred_element_type=jnp.float32)
                ts.append(acc.reshape(th, Wp, cb))
            out = ts[0][:, 0:W] + ts[1][:, 1:W + 1] + ts[2][:, 2:W + 2]
            if n_ci == 1:
                out = out + b_ref[0][None, None, :]
                if act == 'leaky':
                    out = _leaky(out)
                if scale != 1.0:
                    out = out * scale
                o_ref[b, h0:h0 + th] = out
            else:
                @pl.when(ci == 0)
                def _init():
                    o_ref[b, h0:h0 + th] = out

                @pl.when(ci > 0)
                def _accum():
                    o_ref[b, h0:h0 + th] += out
    if n_ci > 1:
        @pl.when(ci == n_ci - 1)
        def _epilogue():
            for b in range(nb):
                y = o_ref[b] + b_ref[0][None, None, :]
                if act == 'leaky':
                    y = _leaky(y)
                if scale != 1.0:
                    y = y * scale
                o_ref[b] = y


def _round_up(v, m):
    return ((v + m - 1) // m) * m


def conv3x3_p(x, w, b, mode, act=None, scale=1.0, cb=128, cib=None, th=16,
              batch_in_grid=False):
    B, H, W, Cin = x.shape
    Cout = w.shape[0]
    th = min(th, H)
    if mode == 'zero':
        xp = jnp.pad(x, ((0, 0), (1, 1), (1, 1), (0, 0)))
    else:
        xp = jnp.pad(x, ((0, 0), (1, 1), (1, 1), (0, 0)), mode=mode)
    Wp = _round_up(W + 2, 8)
    co_pad = _round_up(Cout, cb)
    n_co = co_pad // cb
    if cib is None:
        cib = _round_up(Cin, 128)
    ci_pad = _round_up(Cin, cib)
    n_ci = ci_pad // cib
    xp = jnp.pad(xp, ((0, 0), (0, 0), (0, Wp - (W + 2)),
                      (0, ci_pad - Cin)))
    wt = jnp.transpose(w, (2, 3, 1, 0))
    wt = jnp.pad(wt, ((0, 0), (0, 0), (0, ci_pad - Cin), (0, co_pad - Cout)))
    bp = jnp.pad(b, (0, co_pad - Cout)).reshape(1, co_pad)

    if batch_in_grid:
        grid = (B, n_co, n_ci)
        nb = 1
        xmap = lambda bi, co, ci: (bi, 0, 0, ci)
        wmap = lambda bi, co, ci: (0, 0, ci, co)
        bmap = lambda bi, co, ci: (0, co)
        omap = lambda bi, co, ci: (bi, 0, 0, co)
        ci_axis = 2
    else:
        grid = (n_co, n_ci)
        nb = B
        xmap = lambda co, ci: (0, 0, 0, ci)
        wmap = lambda co, ci: (0, 0, ci, co)
        bmap = lambda co, ci: (0, co)
        omap = lambda co, ci: (0, 0, 0, co)
        ci_axis = 1

    body = functools.partial(_conv_body_impl, H=H, W=W, Wp=Wp, th=th,
                             act=act, scale=scale, n_ci=n_ci, ci_axis=ci_axis)
    out = pl.pallas_call(
        body,
        grid=grid,
        in_specs=[
            pl.BlockSpec((nb, H + 2, Wp, cib), xmap),
            pl.BlockSpec((3, 3, cib, cb), wmap),
            pl.BlockSpec((1, cb), bmap),
        ],
        out_specs=pl.BlockSpec((nb, H, W, cb), omap),
        out_shape=jax.ShapeDtypeStruct((B, H, W, co_pad), jnp.float32),
    )(xp, wt, bp)
    return out[..., :Cout]


def _t_body(x_ref, o_ref):
    o_ref[0] = jnp.transpose(x_ref[0], (1, 2, 0))


def _nhwc_p(x):
    B, C, H, W = x.shape
    return pl.pallas_call(
        _t_body,
        grid=(B,),
        in_specs=[pl.BlockSpec((1, C, H, W), lambda bi: (bi, 0, 0, 0))],
        out_specs=pl.BlockSpec((1, H, W, C), lambda bi: (bi, 0, 0, 0)),
        out_shape=jax.ShapeDtypeStruct((B, H, W, C), jnp.float32),
    )(x)


def _up2(x):
    return jnp.repeat(jnp.repeat(x, 2, axis=1), 2, axis=2)


def _maxpool(x, k):
    return jax.lax.reduce_window(x, -jnp.inf, jax.lax.max,
                                 (1, 1, k, k), (1, 1, 1, 1), 'SAME')


def _iwt_haar(ll, h):
    lh, hl, hh = h[:, :, 0], h[:, :, 1], h[:, :, 2]
    x00 = (ll - lh - hl + hh) * 0.5
    x01 = (ll - lh + hl - hh) * 0.5
    x10 = (ll + lh - hl - hh) * 0.5
    x11 = (ll + lh + hl + hh) * 0.5
    B, C, H, W = ll.shape
    out = jnp.zeros((B, C, 2 * H, 2 * W), ll.dtype)
    out = out.at[:, :, 0::2, 0::2].set(x00)
    out = out.at[:, :, 0::2, 1::2].set(x01)
    out = out.at[:, :, 1::2, 0::2].set(x10)
    out = out.at[:, :, 1::2, 1::2].set(x11)
    return out


def _nhwc(x):
    return jnp.transpose(x, (0, 2, 3, 1))


def _nchw(x):
    return jnp.transpose(x, (0, 3, 1, 2))


def kernel(x_block_0, x_block_1, x_block_2, x_block_3,
           c2w, c2b, u1w, u1b, w1llw, w1llb, w1w, w1b,
           u2w, u2b, w2w, w2b, u3w, u3b, w3w, w3b):
    thresh_ratio = 0.1
    xb4, xb3, xb2, xb1 = x_block_0, x_block_1, x_block_2, x_block_3
    x1 = _nhwc_p(xb1)
    x_d0 = conv3x3_p(x1, c2w, c2b, 'edge', cb=384, cib=768)
    cat1 = jnp.concatenate([_up2(x_d0), _nhwc_p(xb2)], -1)
    x_d1 = conv3x3_p(cat1, u1w, u1b, 'reflect', act='leaky',
                     cb=256, cib=768)
    ll_n = conv3x3_p(x_d1, w1llw, w1llb, 'edge', scale=8.0)
    disp3 = _nchw(ll_n) / 8.0
    h_n = conv3x3_p(x_d1, w1w, w1b, 'zero', scale=4.0)
    ll = _nchw(ll_n)
    h = _nchw(h_n)[:, None]
    ll = _iwt_haar(ll, h)
    disp2 = ll / 4.0

    thresh = (ll.max() - ll.min()) * thresh_ratio
    mask = (jnp.abs(h).max(axis=2) > thresh).astype(jnp.float32)
    up_mask = (_maxpool(mask, 5) > 0).astype(jnp.float32)
    conva_mask = (_maxpool(_up2_nchw(mask), 5) > 0).astype(jnp.float32)
    wave_mask = (_maxpool(_up2_nchw(mask), 3) > 0).astype(jnp.float32)
    wavelet_mask = _up2_nchw(mask)

    xv = x_d1 * _nhwc(up_mask)
    cat2 = jnp.concatenate([_up2(xv), _nhwc_p(xb3)], -1) * _nhwc(conva_mask)
    xv = conv3x3_p(cat2, u2w, u2b, 'reflect', act='leaky',
                   cb=128, cib=384) * _nhwc(wave_mask)
    h2 = conv3x3_p(xv, w2w, w2b, 'zero', scale=2.0) * _nhwc(wavelet_mask)
    h = _nchw(h2)[:, None]
    ll = _iwt_haar(ll, wavelet_mask[:, :, None] * h)
    disp1 = ll / 2.0

    thresh = (ll.max() - ll.min()) * thresh_ratio
    mask = (jnp.abs(h).max(axis=2) > thresh).astype(jnp.float32)
    up_mask = (_maxpool(mask, 5) > 0).astype(jnp.float32)
    conva_mask = (_maxpool(_up2_nchw(mask), 5) > 0).astype(jnp.float32)
    wave_mask = (_maxpool(_up2_nchw(mask), 3) > 0).astype(jnp.float32)
    wavelet_mask = _up2_nchw(mask)

    xv = xv * _nhwc(up_mask)
    cat3 = jnp.concatenate([_up2(xv), _nhwc_p(xb4)], -1) * _nhwc(conva_mask)
    xv = conv3x3_p(cat3, u3w, u3b, 'reflect', act='leaky',
                   cb=128, cib=128, batch_in_grid=True) * _nhwc(wave_mask)
    h3 = conv3x3_p(xv, w3w, w3b, 'zero') * _nhwc(wavelet_mask)
    h = _nchw(h3)[:, None]
    ll = _iwt_haar(ll, wavelet_mask[:, :, None] * h)
    disp0 = ll
    return disp3, disp2, disp1, disp0


def _up2_nchw(x):
    return jnp.repeat(jnp.repeat(x, 2, axis=2), 2, axis=3)


# speedup vs baseline: 1.2046x; 1.1200x over previous
import functools

import jax
import jax.numpy as jnp
from jax.experimental import pallas as pl


def _leaky(x):
    return jnp.where(x >= 0, x, 0.2 * x)


def _conv_body_impl(x_ref, w_ref, b_ref, o_ref, *, H, W, Wp, th, act, scale,
                    n_ci, ci_axis):
    cb = w_ref.shape[-1]
    cib = w_ref.shape[-2]
    nb = x_ref.shape[0]
    ci = pl.program_id(ci_axis)
    for b in range(nb):
        for h0 in range(0, H, th):
            ts = []
            for dx in range(3):
                acc = jnp.zeros((th * Wp, cb), jnp.float32)
                for dy in range(3):
                    xs = x_ref[b, h0 + dy:h0 + dy + th]
                    acc = acc + jnp.dot(xs.reshape(th * Wp, cib),
                                        w_ref[dy, dx],
                                        preferred_element_type=jnp.float32)
                ts.append(acc.reshape(th, Wp, cb))
            out = ts[0][:, 0:W] + ts[1][:, 1:W + 1] + ts[2][:, 2:W + 2]
            if n_ci == 1:
                out = out + b_ref[0][None, None, :]
                if act == 'leaky':
                    out = _leaky(out)
                if scale != 1.0:
                    out = out * scale
                o_ref[b, h0:h0 + th] = out
            else:
                @pl.when(ci == 0)
                def _init():
                    o_ref[b, h0:h0 + th] = out

                @pl.when(ci > 0)
                def _accum():
                    o_ref[b, h0:h0 + th] += out
    if n_ci > 1:
        @pl.when(ci == n_ci - 1)
        def _epilogue():
            for b in range(nb):
                y = o_ref[b] + b_ref[0][None, None, :]
                if act == 'leaky':
                    y = _leaky(y)
                if scale != 1.0:
                    y = y * scale
                o_ref[b] = y


def _round_up(v, m):
    return ((v + m - 1) // m) * m


def conv3x3_p(x, w, b, mode, act=None, scale=1.0, cb=128, cib=None, th=16,
              batch_in_grid=False):
    B, H, W, Cin = x.shape
    Cout = w.shape[0]
    th = min(th, H)
    if mode == 'zero':
        xp = jnp.pad(x, ((0, 0), (1, 1), (1, 1), (0, 0)))
    else:
        xp = jnp.pad(x, ((0, 0), (1, 1), (1, 1), (0, 0)), mode=mode)
    Wp = _round_up(W + 2, 8)
    co_pad = _round_up(Cout, cb)
    n_co = co_pad // cb
    if cib is None:
        cib = _round_up(Cin, 128)
    ci_pad = _round_up(Cin, cib)
    n_ci = ci_pad // cib
    xp = jnp.pad(xp, ((0, 0), (0, 0), (0, Wp - (W + 2)),
                      (0, ci_pad - Cin)))
    wt = jnp.transpose(w, (2, 3, 1, 0))
    wt = jnp.pad(wt, ((0, 0), (0, 0), (0, ci_pad - Cin), (0, co_pad - Cout)))
    bp = jnp.pad(b, (0, co_pad - Cout)).reshape(1, co_pad)

    if batch_in_grid:
        grid = (B, n_co, n_ci)
        nb = 1
        xmap = lambda bi, co, ci: (bi, 0, 0, ci)
        wmap = lambda bi, co, ci: (0, 0, ci, co)
        bmap = lambda bi, co, ci: (0, co)
        omap = lambda bi, co, ci: (bi, 0, 0, co)
        ci_axis = 2
    else:
        grid = (n_co, n_ci)
        nb = B
        xmap = lambda co, ci: (0, 0, 0, ci)
        wmap = lambda co, ci: (0, 0, ci, co)
        bmap = lambda co, ci: (0, co)
        omap = lambda co, ci: (0, 0, 0, co)
        ci_axis = 1

    body = functools.partial(_conv_body_impl, H=H, W=W, Wp=Wp, th=th,
                             act=act, scale=scale, n_ci=n_ci, ci_axis=ci_axis)
    out = pl.pallas_call(
        body,
        grid=grid,
        in_specs=[
            pl.BlockSpec((nb, H + 2, Wp, cib), xmap),
            pl.BlockSpec((3, 3, cib, cb), wmap),
            pl.BlockSpec((1, cb), bmap),
        ],
        out_specs=pl.BlockSpec((nb, H, W, cb), omap),
        out_shape=jax.ShapeDtypeStruct((B, H, W, co_pad), jnp.float32),
    )(xp, wt, bp)
    return out[..., :Cout]


def _dup_rows(t):
    return jnp.concatenate([t[:, None], t[:, None]], axis=1).reshape(
        2 * t.shape[0], t.shape[1], t.shape[2])


def _upconv_body(xa_ref, xs_ref, w_ref, b_ref, o_ref, *, H, W, Wp, th,
                 act, scale, n_ci_a, n_ci, ci_axis):
    cb = w_ref.shape[-1]
    cib = w_ref.shape[-2]
    nb = xs_ref.shape[0]
    ci = pl.program_id(ci_axis)

    def store(b, h0, out):
        @pl.when(ci == 0)
        def _init():
            o_ref[b, h0:h0 + th] = out

        @pl.when(ci > 0)
        def _accum():
            o_ref[b, h0:h0 + th] += out

    @pl.when(ci < n_ci_a)
    def _a_path():
        for b in range(nb):
            for h0 in range(0, H, th):
                ts = []
                for dx in range(3):
                    acc = jnp.zeros((th, Wp, cb), jnp.float32)
                    for dy in range(3):
                        k0 = (h0 + dy + 1) // 2
                        even = (h0 + dy + 1) % 2 == 0
                        m = th // 2 if even else th // 2 + 1
                        xs = xa_ref[b, k0:k0 + m]
                        t = jnp.dot(xs.reshape(m * Wp, cib), w_ref[dy, dx],
                                    preferred_element_type=jnp.float32)
                        t = t.reshape(m, Wp, cb)
                        if even:
                            tf = _dup_rows(t)
                        else:
                            tf = jnp.concatenate(
                                [t[0:1], _dup_rows(t[1:m])[:th - 1]], axis=0)
                        acc = acc + tf
                    ts.append(acc)
                out = ts[0][:, 0:W] + ts[1][:, 1:W + 1] + ts[2][:, 2:W + 2]
                store(b, h0, out)

    @pl.when(ci >= n_ci_a)
    def _s_path():
        for b in range(nb):
            for h0 in range(0, H, th):
                ts = []
                for dx in range(3):
                    acc = jnp.zeros((th * Wp, cb), jnp.float32)
                    for dy in range(3):
                        xs = xs_ref[b, h0 + dy:h0 + dy + th]
                        acc = acc + jnp.dot(xs.reshape(th * Wp, cib),
                                            w_ref[dy, dx],
                                            preferred_element_type=jnp.float32)
                    ts.append(acc.reshape(th, Wp, cb))
                out = ts[0][:, 0:W] + ts[1][:, 1:W + 1] + ts[2][:, 2:W + 2]
                store(b, h0, out)

    @pl.when(ci == n_ci - 1)
    def _epilogue():
        for b in range(nb):
            y = o_ref[b] + b_ref[0][None, None, :]
            if act == 'leaky':
                y = _leaky(y)
            if scale != 1.0:
                y = y * scale
            o_ref[b] = y


def conv3x3_up_p(a, s, w, b, act=None, scale=1.0, cb=128, cib=128, th=16,
                 batch_in_grid=False):
    B, Hh, Wh, C1 = a.shape
    _, H, W, C2 = s.shape
    Cout = w.shape[0]
    ar = jnp.concatenate([a[:, :1], a, a[:, -1:]], axis=1)
    au = jnp.concatenate([ar[:, :, :1], jnp.repeat(ar, 2, axis=2),
                          ar[:, :, -1:]], axis=2)
    sp = jnp.pad(s, ((0, 0), (1, 1), (1, 1), (0, 0)), mode='reflect')
    Wp = _round_up(W + 2, 8)
    cipa = _round_up(C1, cib)
    cips = _round_up(C2, cib)
    n_ci_a = cipa // cib
    n_ci_s = cips // cib
    n_ci = n_ci_a + n_ci_s
    co_pad = _round_up(Cout, cb)
    n_co = co_pad // cb
    au = jnp.pad(au, ((0, 0), (0, 0), (0, Wp - (W + 2)), (0, cipa - C1)))
    sp = jnp.pad(sp, ((0, 0), (0, 0), (0, Wp - (W + 2)), (0, cips - C2)))
    wa = jnp.pad(jnp.transpose(w[:, :C1], (2, 3, 1, 0)),
                 ((0, 0), (0, 0), (0, cipa - C1), (0, co_pad - Cout)))
    ws = jnp.pad(jnp.transpose(w[:, C1:], (2, 3, 1, 0)),
                 ((0, 0), (0, 0), (0, cips - C2), (0, co_pad - Cout)))
    wt = jnp.concatenate([wa, ws], axis=2)
    bp = jnp.pad(b, (0, co_pad - Cout)).reshape(1, co_pad)

    if batch_in_grid:
        grid = (B, n_co, n_ci)
        nb = 1
        amap = lambda bi, co, ci: (bi, 0, 0, _imin(ci, n_ci_a - 1))
        smap = lambda bi, co, ci: (bi, 0, 0, _imax(ci - n_ci_a, 0))
        wmap = lambda bi, co, ci: (0, 0, ci, co)
        bmap = lambda bi, co, ci: (0, co)
        omap = lambda bi, co, ci: (bi, 0, 0, co)
        ci_axis = 2
    else:
        grid = (n_co, n_ci)
        nb = B
        amap = lambda co, ci: (0, 0, 0, _imin(ci, n_ci_a - 1))
        smap = lambda co, ci: (0, 0, 0, _imax(ci - n_ci_a, 0))
        wmap = lambda co, ci: (0, 0, ci, co)
        bmap = lambda co, ci: (0, co)
        omap = lambda co, ci: (0, 0, 0, co)
        ci_axis = 1

    body = functools.partial(_upconv_body, H=H, W=W, Wp=Wp, th=min(th, H),
                             act=act, scale=scale, n_ci_a=n_ci_a, n_ci=n_ci,
                             ci_axis=ci_axis)
    out = pl.pallas_call(
        body,
        grid=grid,
        in_specs=[
            pl.BlockSpec((nb, Hh + 2, Wp, cib), amap),
            pl.BlockSpec((nb, H + 2, Wp, cib), smap),
            pl.BlockSpec((3, 3, cib, cb), wmap),
            pl.BlockSpec((1, cb), bmap),
        ],
        out_specs=pl.BlockSpec((nb, H, W, cb), omap),
        out_shape=jax.ShapeDtypeStruct((B, H, W, co_pad), jnp.float32),
    )(au, sp, wt, bp)
    return out[..., :Cout]


def _imin(x, y):
    return jnp.minimum(x, y)


def _imax(x, y):
    return jnp.maximum(x, y)


def _t_body(x_ref, o_ref):
    o_ref[0] = jnp.transpose(x_ref[0], (1, 2, 0))


def _nhwc_p(x):
    B, C, H, W = x.shape
    return pl.pallas_call(
        _t_body,
        grid=(B,),
        in_specs=[pl.BlockSpec((1, C, H, W), lambda bi: (bi, 0, 0, 0))],
        out_specs=pl.BlockSpec((1, H, W, C), lambda bi: (bi, 0, 0, 0)),
        out_shape=jax.ShapeDtypeStruct((B, H, W, C), jnp.float32),
    )(x)


def _up2(x):
    return jnp.repeat(jnp.repeat(x, 2, axis=1), 2, axis=2)


def _maxpool(x, k):
    return jax.lax.reduce_window(x, -jnp.inf, jax.lax.max,
                                 (1, 1, k, k), (1, 1, 1, 1), 'SAME')


def _iwt_haar(ll, h):
    lh, hl, hh = h[:, :, 0], h[:, :, 1], h[:, :, 2]
    x00 = (ll - lh - hl + hh) * 0.5
    x01 = (ll - lh + hl - hh) * 0.5
    x10 = (ll + lh - hl - hh) * 0.5
    x11 = (ll + lh + hl + hh) * 0.5
    B, C, H, W = ll.shape
    out = jnp.zeros((B, C, 2 * H, 2 * W), ll.dtype)
    out = out.at[:, :, 0::2, 0::2].set(x00)
    out = out.at[:, :, 0::2, 1::2].set(x01)
    out = out.at[:, :, 1::2, 0::2].set(x10)
    out = out.at[:, :, 1::2, 1::2].set(x11)
    return out


def _nhwc(x):
    return jnp.transpose(x, (0, 2, 3, 1))


def _mask_hwc(m):
    B, _, H, W = m.shape
    return m.reshape(B, H, W, 1)


def _nchw(x):
    return jnp.transpose(x, (0, 3, 1, 2))


def kernel(x_block_0, x_block_1, x_block_2, x_block_3,
           c2w, c2b, u1w, u1b, w1llw, w1llb, w1w, w1b,
           u2w, u2b, w2w, w2b, u3w, u3b, w3w, w3b):
    thresh_ratio = 0.1
    xb4, xb3, xb2, xb1 = x_block_0, x_block_1, x_block_2, x_block_3
    x1 = _nhwc_p(xb1)
    x_d0 = conv3x3_p(x1, c2w, c2b, 'edge', cb=384, cib=768)
    x_d1 = conv3x3_up_p(x_d0, _nhwc_p(xb2), u1w, u1b, act='leaky',
                        cb=256, cib=384)
    ll_n = conv3x3_p(x_d1, w1llw, w1llb, 'edge', scale=8.0)
    disp3 = _nchw(ll_n) / 8.0
    h_n = conv3x3_p(x_d1, w1w, w1b, 'zero', scale=4.0)
    ll = _nchw(ll_n)
    h = _nchw(h_n)[:, None]
    ll = _iwt_haar(ll, h)
    disp2 = ll / 4.0

    thresh = (ll.max() - ll.min()) * thresh_ratio
    mask = (jnp.abs(h).max(axis=2) > thresh).astype(jnp.float32)
    wave_mask = (_maxpool(_up2_nchw(mask), 3) > 0).astype(jnp.float32)
    wavelet_mask = _up2_nchw(mask)

    xv = conv3x3_up_p(x_d1, _nhwc_p(xb3), u2w, u2b, act='leaky',
                      cb=128, cib=128) * _mask_hwc(wave_mask)
    h2 = conv3x3_p(xv, w2w, w2b, 'zero', scale=2.0) * _mask_hwc(wavelet_mask)
    h = _nchw(h2)[:, None]
    ll = _iwt_haar(ll, wavelet_mask[:, :, None] * h)
    disp1 = ll / 2.0

    thresh = (ll.max() - ll.min()) * thresh_ratio
    mask = (jnp.abs(h).max(axis=2) > thresh).astype(jnp.float32)
    wave_mask = (_maxpool(_up2_nchw(mask), 3) > 0).astype(jnp.float32)
    wavelet_mask = _up2_nchw(mask)

    xv = conv3x3_up_p(xv, _nhwc_p(xb4), u3w, u3b, act='leaky',
                      cb=128, cib=128,
                      batch_in_grid=True) * _mask_hwc(wave_mask)
    h3 = conv3x3_p(xv, w3w, w3b, 'zero') * _mask_hwc(wavelet_mask)
    h = _nchw(h3)[:, None]
    ll = _iwt_haar(ll, wavelet_mask[:, :, None] * h)
    disp0 = ll
    return disp3, disp2, disp1, disp0


def _up2_nchw(x):
    return jnp.repeat(jnp.repeat(x, 2, axis=2), 2, axis=3)


# speedup vs baseline: 1.2125x; 1.0066x over previous
import functools

import jax
import jax.numpy as jnp
from jax.experimental import pallas as pl


def _leaky(x):
    return jnp.where(x >= 0, x, 0.2 * x)


def _conv_body_impl(x_ref, w_ref, b_ref, o_ref, *, H, W, Wp, th, act, scale,
                    n_ci, ci_axis):
    cb = w_ref.shape[-1]
    cib = w_ref.shape[-2]
    nb = x_ref.shape[0]
    ci = pl.program_id(ci_axis)
    for b in range(nb):
        for h0 in range(0, H, th):
            ts = []
            for dx in range(3):
                acc = jnp.zeros((th * Wp, cb), jnp.float32)
                for dy in range(3):
                    xs = x_ref[b, h0 + dy:h0 + dy + th]
                    acc = acc + jnp.dot(xs.reshape(th * Wp, cib),
                                        w_ref[dy, dx],
                                        preferred_element_type=jnp.float32)
                ts.append(acc.reshape(th, Wp, cb))
            out = ts[0][:, 0:W] + ts[1][:, 1:W + 1] + ts[2][:, 2:W + 2]
            if n_ci == 1:
                out = out + b_ref[0][None, None, :]
                if act == 'leaky':
                    out = _leaky(out)
                if scale != 1.0:
                    out = out * scale
                o_ref[b, h0:h0 + th] = out
            else:
                @pl.when(ci == 0)
                def _init():
                    o_ref[b, h0:h0 + th] = out

                @pl.when(ci > 0)
                def _accum():
                    o_ref[b, h0:h0 + th] += out
    if n_ci > 1:
        @pl.when(ci == n_ci - 1)
        def _epilogue():
            for b in range(nb):
                y = o_ref[b] + b_ref[0][None, None, :]
                if act == 'leaky':
                    y = _leaky(y)
                if scale != 1.0:
                    y = y * scale
                o_ref[b] = y


def _round_up(v, m):
    return ((v + m - 1) // m) * m


def conv3x3_p(x, w, b, mode, act=None, scale=1.0, cb=128, cib=None, th=16,
              batch_in_grid=False):
    B, H, W, Cin = x.shape
    Cout = w.shape[0]
    th = min(th, H)
    if mode == 'zero':
        xp = jnp.pad(x, ((0, 0), (1, 1), (1, 1), (0, 0)))
    else:
        xp = jnp.pad(x, ((0, 0), (1, 1), (1, 1), (0, 0)), mode=mode)
    Wp = _round_up(W + 2, 8)
    co_pad = _round_up(Cout, cb)
    n_co = co_pad // cb
    if cib is None:
        cib = _round_up(Cin, 128)
    ci_pad = _round_up(Cin, cib)
    n_ci = ci_pad // cib
    xp = jnp.pad(xp, ((0, 0), (0, 0), (0, Wp - (W + 2)),
                      (0, ci_pad - Cin)))
    wt = jnp.transpose(w, (2, 3, 1, 0))
    wt = jnp.pad(wt, ((0, 0), (0, 0), (0, ci_pad - Cin), (0, co_pad - Cout)))
    bp = jnp.pad(b, (0, co_pad - Cout)).reshape(1, co_pad)

    if batch_in_grid:
        grid = (B, n_co, n_ci)
        nb = 1
        xmap = lambda bi, co, ci: (bi, 0, 0, ci)
        wmap = lambda bi, co, ci: (0, 0, ci, co)
        bmap = lambda bi, co, ci: (0, co)
        omap = lambda bi, co, ci: (bi, 0, 0, co)
        ci_axis = 2
    else:
        grid = (n_co, n_ci)
        nb = B
        xmap = lambda co, ci: (0, 0, 0, ci)
        wmap = lambda co, ci: (0, 0, ci, co)
        bmap = lambda co, ci: (0, co)
        omap = lambda co, ci: (0, 0, 0, co)
        ci_axis = 1

    body = functools.partial(_conv_body_impl, H=H, W=W, Wp=Wp, th=th,
                             act=act, scale=scale, n_ci=n_ci, ci_axis=ci_axis)
    out = pl.pallas_call(
        body,
        grid=grid,
        in_specs=[
            pl.BlockSpec((nb, H + 2, Wp, cib), xmap),
            pl.BlockSpec((3, 3, cib, cb), wmap),
            pl.BlockSpec((1, cb), bmap),
        ],
        out_specs=pl.BlockSpec((nb, H, W, cb), omap),
        out_shape=jax.ShapeDtypeStruct((B, H, W, co_pad), jnp.float32),
    )(xp, wt, bp)
    return out[..., :Cout]


def _zconv_body(x_ref, mi_ref, mo_ref, w_ref, b_ref, o_ref, *, H, W, th,
                scale, has_mi, has_mo, n_ci, ci_axis):
    cb = w_ref.shape[-1]
    cib = w_ref.shape[-2]
    nb = x_ref.shape[0]
    ci = pl.program_id(ci_axis)
    for b in range(nb):
        for h0 in range(0, H, th):
            acc = jnp.zeros((th, W, cb), jnp.float32)
            for dy in range(3):
                d = dy - 1
                r0 = max(h0 + d, 0)
                r1 = min(h0 + th + d, H)
                m = r1 - r0
                a0 = r0 - d - h0
                xs = x_ref[b, r0:r1]
                if has_mi:
                    xs = xs * mi_ref[b, r0:r1][:, :, None]
                for dx in range(3):
                    t = jnp.dot(xs.reshape(m * W, cib), w_ref[dy, dx],
                                preferred_element_type=jnp.float32)
                    t = t.reshape(m, W, cb)
                    e = dx - 1
                    j0 = max(0, -e)
                    j1 = min(W, W - e)
                    acc = acc + jnp.pad(t[:, j0 + e:j1 + e],
                                        ((a0, th - a0 - m),
                                         (j0, W - j1), (0, 0)))
            if n_ci == 1:
                y = acc + b_ref[0][None, None, :]
                if has_mo:
                    y = y * mo_ref[b, h0:h0 + th][:, :, None]
                if scale != 1.0:
                    y = y * scale
                o_ref[b, h0:h0 + th] = y
            else:
                @pl.when(ci == 0)
                def _init():
                    o_ref[b, h0:h0 + th] = acc

                @pl.when(ci > 0)
                def _accum():
                    o_ref[b, h0:h0 + th] += acc
    if n_ci > 1:
        @pl.when(ci == n_ci - 1)
        def _epi():
            for b in range(nb):
                y = o_ref[b] + b_ref[0][None, None, :]
                if has_mo:
                    y = y * mo_ref[b][:, :, None]
                if scale != 1.0:
                    y = y * scale
                o_ref[b] = y


def zconv_p(x, w, b, mask_in=None, mask_out=None, scale=1.0, cb=128,
            cib=None, th=16, batch_in_grid=False):
    B, H, W, Cx = x.shape
    Cout, Cin = w.shape[0], w.shape[1]
    th = min(th, H)
    if cib is None:
        cib = Cx
    n_ci = Cx // cib
    co_pad = _round_up(Cout, cb)
    n_co = co_pad // cb
    wt = jnp.transpose(w, (2, 3, 1, 0))
    wt = jnp.pad(wt, ((0, 0), (0, 0), (0, Cx - Cin), (0, co_pad - Cout)))
    bp = jnp.pad(b, (0, co_pad - Cout)).reshape(1, co_pad)
    has_mi = mask_in is not None
    has_mo = mask_out is not None
    mi = (mask_in.reshape(B, H, W) if has_mi
          else jnp.zeros((B, H, W), jnp.float32))
    mo = (mask_out.reshape(B, H, W) if has_mo
          else jnp.zeros((B, H, W), jnp.float32))

    if batch_in_grid:
        grid = (B, n_co, n_ci)
        nb = 1
        xmap = lambda bi, co, ci: (bi, 0, 0, ci)
        mmap = lambda bi, co, ci: (bi, 0, 0)
        wmap = lambda bi, co, ci: (0, 0, ci, co)
        bmap = lambda bi, co, ci: (0, co)
        omap = lambda bi, co, ci: (bi, 0, 0, co)
        ci_axis = 2
    else:
        grid = (n_co, n_ci)
        nb = B
        xmap = lambda co, ci: (0, 0, 0, ci)
        mmap = lambda co, ci: (0, 0, 0)
        wmap = lambda co, ci: (0, 0, ci, co)
        bmap = lambda co, ci: (0, co)
        omap = lambda co, ci: (0, 0, 0, co)
        ci_axis = 1

    body = functools.partial(_zconv_body, H=H, W=W, th=th, scale=scale,
                             has_mi=has_mi, has_mo=has_mo, n_ci=n_ci,
                             ci_axis=ci_axis)
    out = pl.pallas_call(
        body,
        grid=grid,
        in_specs=[
            pl.BlockSpec((nb, H, W, cib), xmap),
            pl.BlockSpec((nb, H, W), mmap),
            pl.BlockSpec((nb, H, W), mmap),
            pl.BlockSpec((3, 3, cib, cb), wmap),
            pl.BlockSpec((1, cb), bmap),
        ],
        out_specs=pl.BlockSpec((nb, H, W, cb), omap),
        out_shape=jax.ShapeDtypeStruct((B, H, W, co_pad), jnp.float32),
    )(x, mi, mo, wt, bp)
    return out[..., :Cout]


def _dup_rows(t):
    return jnp.concatenate([t[:, None], t[:, None]], axis=1).reshape(
        2 * t.shape[0], t.shape[1], t.shape[2])


def _upconv_body(xa_ref, xs_ref, w_ref, b_ref, o_ref, *, H, W, Wp, th,
                 act, scale, n_ci_a, n_ci, ci_axis):
    cb = w_ref.shape[-1]
    cib = w_ref.shape[-2]
    nb = xs_ref.shape[0]
    ci = pl.program_id(ci_axis)

    def store(b, h0, out):
        @pl.when(ci == 0)
        def _init():
            o_ref[b, h0:h0 + th] = out

        @pl.when(ci > 0)
        def _accum():
            o_ref[b, h0:h0 + th] += out

    @pl.when(ci < n_ci_a)
    def _a_path():
        for b in range(nb):
            for h0 in range(0, H, th):
                ts = []
                for dx in range(3):
                    acc = jnp.zeros((th, Wp, cb), jnp.float32)
                    for dy in range(3):
                        k0 = (h0 + dy + 1) // 2
                        even = (h0 + dy + 1) % 2 == 0
                        m = th // 2 if even else th // 2 + 1
                        xs = xa_ref[b, k0:k0 + m]
                        t = jnp.dot(xs.reshape(m * Wp, cib), w_ref[dy, dx],
                                    preferred_element_type=jnp.float32)
                        t = t.reshape(m, Wp, cb)
                        if even:
                            tf = _dup_rows(t)
                        else:
                            tf = jnp.concatenate(
                                [t[0:1], _dup_rows(t[1:m])[:th - 1]], axis=0)
                        acc = acc + tf
                    ts.append(acc)
                out = ts[0][:, 0:W] + ts[1][:, 1:W + 1] + ts[2][:, 2:W + 2]
                store(b, h0, out)

    @pl.when(ci >= n_ci_a)
    def _s_path():
        for b in range(nb):
            for h0 in range(0, H, th):
                ts = []
                for dx in range(3):
                    acc = jnp.zeros((th * Wp, cb), jnp.float32)
                    for dy in range(3):
                        xs = xs_ref[b, h0 + dy:h0 + dy + th]
                        acc = acc + jnp.dot(xs.reshape(th * Wp, cib),
                                            w_ref[dy, dx],
                                            preferred_element_type=jnp.float32)
                    ts.append(acc.reshape(th, Wp, cb))
                out = ts[0][:, 0:W] + ts[1][:, 1:W + 1] + ts[2][:, 2:W + 2]
                store(b, h0, out)

    @pl.when(ci == n_ci - 1)
    def _epilogue():
        for b in range(nb):
            y = o_ref[b] + b_ref[0][None, None, :]
            if act == 'leaky':
                y = _leaky(y)
            if scale != 1.0:
                y = y * scale
            o_ref[b] = y


def conv3x3_up_p(a, s, w, b, act=None, scale=1.0, cb=128, cib=128, th=16,
                 batch_in_grid=False, raw=False):
    B, Hh, Wh, C1 = a.shape
    _, H, W, C2 = s.shape
    Cout = w.shape[0]
    ar = jnp.concatenate([a[:, :1], a, a[:, -1:]], axis=1)
    au = jnp.concatenate([ar[:, :, :1], jnp.repeat(ar, 2, axis=2),
                          ar[:, :, -1:]], axis=2)
    sp = jnp.pad(s, ((0, 0), (1, 1), (1, 1), (0, 0)), mode='reflect')
    Wp = _round_up(W + 2, 8)
    cipa = _round_up(C1, cib)
    cips = _round_up(C2, cib)
    n_ci_a = cipa // cib
    n_ci_s = cips // cib
    n_ci = n_ci_a + n_ci_s
    co_pad = _round_up(Cout, cb)
    n_co = co_pad // cb
    au = jnp.pad(au, ((0, 0), (0, 0), (0, Wp - (W + 2)), (0, cipa - C1)))
    sp = jnp.pad(sp, ((0, 0), (0, 0), (0, Wp - (W + 2)), (0, cips - C2)))
    wa = jnp.pad(jnp.transpose(w[:, :C1], (2, 3, 1, 0)),
                 ((0, 0), (0, 0), (0, cipa - C1), (0, co_pad - Cout)))
    ws = jnp.pad(jnp.transpose(w[:, C1:], (2, 3, 1, 0)),
                 ((0, 0), (0, 0), (0, cips - C2), (0, co_pad - Cout)))
    wt = jnp.concatenate([wa, ws], axis=2)
    bp = jnp.pad(b, (0, co_pad - Cout)).reshape(1, co_pad)

    if batch_in_grid:
        grid = (B, n_co, n_ci)
        nb = 1
        amap = lambda bi, co, ci: (bi, 0, 0, _imin(ci, n_ci_a - 1))
        smap = lambda bi, co, ci: (bi, 0, 0, _imax(ci - n_ci_a, 0))
        wmap = lambda bi, co, ci: (0, 0, ci, co)
        bmap = lambda bi, co, ci: (0, co)
        omap = lambda bi, co, ci: (bi, 0, 0, co)
        ci_axis = 2
    else:
        grid = (n_co, n_ci)
        nb = B
        amap = lambda co, ci: (0, 0, 0, _imin(ci, n_ci_a - 1))
        smap = lambda co, ci: (0, 0, 0, _imax(ci - n_ci_a, 0))
        wmap = lambda co, ci: (0, 0, ci, co)
        bmap = lambda co, ci: (0, co)
        omap = lambda co, ci: (0, 0, 0, co)
        ci_axis = 1

    body = functools.partial(_upconv_body, H=H, W=W, Wp=Wp, th=min(th, H),
                             act=act, scale=scale, n_ci_a=n_ci_a, n_ci=n_ci,
                             ci_axis=ci_axis)
    out = pl.pallas_call(
        body,
        grid=grid,
        in_specs=[
            pl.BlockSpec((nb, Hh + 2, Wp, cib), amap),
            pl.BlockSpec((nb, H + 2, Wp, cib), smap),
            pl.BlockSpec((3, 3, cib, cb), wmap),
            pl.BlockSpec((1, cb), bmap),
        ],
        out_specs=pl.BlockSpec((nb, H, W, cb), omap),
        out_shape=jax.ShapeDtypeStruct((B, H, W, co_pad), jnp.float32),
    )(au, sp, wt, bp)
    return out if raw else out[..., :Cout]


def _imin(x, y):
    return jnp.minimum(x, y)


def _imax(x, y):
    return jnp.maximum(x, y)


def _t_body(x_ref, o_ref):
    o_ref[0] = jnp.transpose(x_ref[0], (1, 2, 0))


def _nhwc_p(x):
    B, C, H, W = x.shape
    return pl.pallas_call(
        _t_body,
        grid=(B,),
        in_specs=[pl.BlockSpec((1, C, H, W), lambda bi: (bi, 0, 0, 0))],
        out_specs=pl.BlockSpec((1, H, W, C), lambda bi: (bi, 0, 0, 0)),
        out_shape=jax.ShapeDtypeStruct((B, H, W, C), jnp.float32),
    )(x)


def _up2(x):
    return jnp.repeat(jnp.repeat(x, 2, axis=1), 2, axis=2)


def _maxpool(x, k):
    return jax.lax.reduce_window(x, -jnp.inf, jax.lax.max,
                                 (1, 1, k, k), (1, 1, 1, 1), 'SAME')


def _iwt_haar(ll, h):
    lh, hl, hh = h[:, :, 0], h[:, :, 1], h[:, :, 2]
    x00 = (ll - lh - hl + hh) * 0.5
    x01 = (ll - lh + hl - hh) * 0.5
    x10 = (ll + lh - hl - hh) * 0.5
    x11 = (ll + lh + hl + hh) * 0.5
    B, C, H, W = ll.shape
    out = jnp.zeros((B, C, 2 * H, 2 * W), ll.dtype)
    out = out.at[:, :, 0::2, 0::2].set(x00)
    out = out.at[:, :, 0::2, 1::2].set(x01)
    out = out.at[:, :, 1::2, 0::2].set(x10)
    out = out.at[:, :, 1::2, 1::2].set(x11)
    return out


def _nhwc(x):
    return jnp.transpose(x, (0, 2, 3, 1))


def _mask_hwc(m):
    B, _, H, W = m.shape
    return m.reshape(B, H, W, 1)


def _nchw(x):
    return jnp.transpose(x, (0, 3, 1, 2))


def kernel(x_block_0, x_block_1, x_block_2, x_block_3,
           c2w, c2b, u1w, u1b, w1llw, w1llb, w1w, w1b,
           u2w, u2b, w2w, w2b, u3w, u3b, w3w, w3b):
    thresh_ratio = 0.1
    xb4, xb3, xb2, xb1 = x_block_0, x_block_1, x_block_2, x_block_3
    x1 = _nhwc_p(xb1)
    x_d0 = conv3x3_p(x1, c2w, c2b, 'edge', cb=384, cib=768)
    x_d1 = conv3x3_up_p(x_d0, _nhwc_p(xb2), u1w, u1b, act='leaky',
                        cb=256, cib=384)
    ll_n = conv3x3_p(x_d1, w1llw, w1llb, 'edge', scale=8.0)
    disp3 = _nchw(ll_n) / 8.0
    h_n = zconv_p(x_d1, w1w, w1b, scale=4.0)[..., :3]
    ll = _nchw(ll_n)
    h = _nchw(h_n)[:, None]
    ll = _iwt_haar(ll, h)
    disp2 = ll / 4.0

    thresh = (ll.max() - ll.min()) * thresh_ratio
    mask = (jnp.abs(h).max(axis=2) > thresh).astype(jnp.float32)
    wave_mask_l1 = (_maxpool(_up2_nchw(mask), 3) > 0).astype(jnp.float32)
    wavelet_mask = _up2_nchw(mask)

    xv_raw = conv3x3_up_p(x_d1, _nhwc_p(xb3), u2w, u2b, act='leaky',
                          cb=128, cib=128, raw=True)
    h2 = zconv_p(xv_raw, w2w, w2b, mask_in=wave_mask_l1, mask_out=wavelet_mask,
                 scale=2.0)[..., :3]
    h = _nchw(h2)[:, None]
    ll = _iwt_haar(ll, wavelet_mask[:, :, None] * h)
    disp1 = ll / 2.0

    thresh = (ll.max() - ll.min()) * thresh_ratio
    mask = (jnp.abs(h).max(axis=2) > thresh).astype(jnp.float32)
    wave_mask = (_maxpool(_up2_nchw(mask), 3) > 0).astype(jnp.float32)
    wavelet_mask = _up2_nchw(mask)

    xv = xv_raw[..., :276] * _mask_hwc(wave_mask_l1)
    xv0_raw = conv3x3_up_p(xv, _nhwc_p(xb4), u3w, u3b, act='leaky',
                           cb=128, cib=128, batch_in_grid=True,
                           raw=True)
    h3 = zconv_p(xv0_raw, w3w, w3b, mask_in=wave_mask, mask_out=wavelet_mask,
                 cib=128, batch_in_grid=True)[..., :3]
    h = _nchw(h3)[:, None]
    ll = _iwt_haar(ll, wavelet_mask[:, :, None] * h)
    disp0 = ll
    return disp3, disp2, disp1, disp0


def _up2_nchw(x):
    return jnp.repeat(jnp.repeat(x, 2, axis=2), 2, axis=3)
